# baseline (speedup 1.0000x reference)
import functools
import math

import jax
import jax.numpy as jnp
from jax import lax
from jax.experimental import pallas as pl
from jax.experimental.pallas import tpu as pltpu
from jax.experimental.pallas import tpu_sc as plsc

NC = 2
NS = 16
NW = NC * NS
BLK = 128


AGG_BG = 64
AGG_NB = 4


@functools.lru_cache(maxsize=None)
def _make_sc_agg(n, d, nbt, n_pad, dtype=jnp.float32):
  bg = AGG_BG
  nb = AGG_NB
  rpt = n_pad // NS
  nzc = rpt // BLK
  hb = nbt // 2
  assert hb % nb == 0
  mesh = plsc.VectorSubcoreMesh(core_axis_name="c", subcore_axis_name="s")

  def body(x_hbm, src_hbm, dst_hbm, agg_out, src_stage, dst_stage,
           *rest):
    rows = rest[:nb]
    agg_sh = rest[nb]
    x_sh = rest[nb + 1]
    sems = rest[nb + 2:]
    c = lax.axis_index("c")
    s = lax.axis_index("s")
    w = c * NS + s

    for k in range(nzc):
      r0 = s * rpt + k * BLK
      pltpu.sync_copy(x_hbm.at[pl.ds(r0, BLK), :], x_sh.at[pl.ds(r0, BLK)])

    lane = 16 if dtype == jnp.float32 else 32
    zv = jnp.zeros((lane,), dtype)
    def zrow(i, carry):
      for q in range(d // lane):
        rows[0][i, pl.ds(q * lane, lane)] = zv
      return carry
    lax.fori_loop(0, bg, zrow, 0)
    for k in range(nzc * (BLK // bg)):
      pltpu.sync_copy(rows[0], agg_sh.at[pl.ds(s * rpt + k * bg, bg)])
    plsc.subcore_barrier()

    for h in range(2):
      base = w * nbt + h * hb
      pltpu.sync_copy(src_hbm.at[pl.ds(base, hb), :], src_stage)
      pltpu.sync_copy(dst_hbm.at[pl.ds(base, hb), :], dst_stage)
      for b in range(nb):
        pltpu.async_copy(x_sh.at[src_stage.at[b]], rows[b], sems[b])

      def group(g, carry):
        j0 = nb * g
        for b in range(nb):
          j = j0 + b
          pltpu.make_async_copy(
              x_sh.at[src_stage.at[j]], rows[b], sems[b]).wait()
          pltpu.sync_copy(rows[b], agg_sh.at[dst_stage.at[j]], add=True)

          @pl.when(j + nb < hb)
          def _():
            pltpu.async_copy(
                x_sh.at[src_stage.at[j + nb]], rows[b], sems[b])
        return carry
      lax.fori_loop(0, hb // nb, group, 0)
    plsc.subcore_barrier()

    for k in range(nzc):
      r0 = s * rpt + k * BLK
      pltpu.sync_copy(agg_sh.at[pl.ds(r0, BLK)],
                      agg_out.at[c, pl.ds(r0, BLK), :])

  return pl.kernel(
      body,
      out_type=jax.ShapeDtypeStruct((NC, n_pad, d), dtype),
      mesh=mesh,
      compiler_params=pltpu.CompilerParams(use_tc_tiling_on_sc=False),
      scratch_types=(
          pltpu.VMEM((hb, bg), jnp.int32),
          pltpu.VMEM((hb, bg), jnp.int32),
          *[pltpu.VMEM((bg, d), dtype) for _ in range(nb)],
          pltpu.VMEM_SHARED((n_pad, d), dtype),
          pltpu.VMEM_SHARED((n_pad, d), dtype),
          *[pltpu.SemaphoreType.DMA for _ in range(nb)],
      ),
  )


@functools.lru_cache(maxsize=None)
def _make_sc_deg(bpt, n_pad, d):
  rpt = n_pad // NS
  nzc = rpt // BLK
  mesh = plsc.VectorSubcoreMesh(core_axis_name="c", subcore_axis_name="s")

  def body(dst_hbm, deg_out, dst_stage, obuf, deg_sh):
    c = lax.axis_index("c")
    s = lax.axis_index("s")
    w = c * NS + s

    pltpu.sync_copy(dst_hbm.at[pl.ds(w * bpt, bpt), :], dst_stage)
    zv = jnp.zeros((16,), jnp.float32)
    def zfill(i, carry):
      for q in range(d // 16):
        obuf[i, pl.ds(q * 16, 16)] = zv
      return carry
    lax.fori_loop(0, BLK, zfill, 0)
    for k in range(nzc):
      pltpu.sync_copy(obuf, deg_sh.at[pl.ds(s * rpt + k * BLK, BLK)])
    ov = jnp.ones((16,), jnp.float32)
    def ofill(i, carry):
      for q in range(d // 16):
        obuf[i, pl.ds(q * 16, 16)] = ov
      return carry
    lax.fori_loop(0, BLK, ofill, 0)
    plsc.subcore_barrier()

    def step(j, carry):
      pltpu.sync_copy(obuf, deg_sh.at[dst_stage.at[j]], add=True)
      return carry
    lax.fori_loop(0, bpt, step, 0)
    plsc.subcore_barrier()

    for k in range(nzc):
      r0 = s * rpt + k * BLK
      pltpu.sync_copy(deg_sh.at[pl.ds(r0, BLK), pl.ds(0, 8)],
                      deg_out.at[c, pl.ds(r0, BLK), :])

  return pl.kernel(
      body,
      out_type=jax.ShapeDtypeStruct((NC, n_pad, 8), jnp.float32),
      mesh=mesh,
      compiler_params=pltpu.CompilerParams(use_tc_tiling_on_sc=False),
      scratch_types=(
          pltpu.VMEM((bpt, BLK), jnp.int32),
          pltpu.VMEM((BLK, d), jnp.float32),
          pltpu.VMEM_SHARED((n_pad, d), jnp.float32),
      ),
  )


@functools.lru_cache(maxsize=None)
def _make_tc_layer(n, d, n_pad, relu, br):
  def body(x_ref, a0_ref, a1_ref, d0_ref, d1_ref, ws_ref, wn_ref, b_ref,
           o_ref):
    deg = (d0_ref[0] + d1_ref[0])[:, :1]
    rdeg = 1.0 / jnp.maximum(deg, 1.0)
    hn = (a0_ref[0].astype(jnp.float32) + a1_ref[0].astype(jnp.float32)) * rdeg
    acc = jnp.dot(x_ref[...], ws_ref[...], preferred_element_type=jnp.float32)
    acc = acc + jnp.dot(hn, wn_ref[...], preferred_element_type=jnp.float32)
    acc = acc + b_ref[...]
    o_ref[...] = jnp.maximum(acc, 0.0) if relu else acc

  return pl.pallas_call(
      body,
      grid=(n // br,),
      in_specs=[
          pl.BlockSpec((br, d), lambda i: (i, 0)),
          pl.BlockSpec((1, br, d), lambda i: (0, i, 0)),
          pl.BlockSpec((1, br, d), lambda i: (1, i, 0)),
          pl.BlockSpec((1, br, 8), lambda i: (0, i, 0)),
          pl.BlockSpec((1, br, 8), lambda i: (1, i, 0)),
          pl.BlockSpec((d, d), lambda i: (0, 0)),
          pl.BlockSpec((d, d), lambda i: (0, 0)),
          pl.BlockSpec((1, d), lambda i: (0, 0)),
      ],
      out_specs=pl.BlockSpec((br, d), lambda i: (i, 0)),
      out_shape=jax.ShapeDtypeStruct((n, d), jnp.float32),
  )


def kernel(g_features, edge_index, W1_self, W1_neigh, b1,
           W2_self, W2_neigh, b2, W3_self, W3_neigh, b3):
  n, d = g_features.shape
  e = edge_index.shape[1]
  quantum = NW * AGG_BG * 2 * AGG_NB
  e_pad = math.ceil(e / quantum) * quantum
  ept = e_pad // NW
  nbt = ept // AGG_BG
  bptd = ept // BLK
  n_pad = math.ceil(n / (NS * BLK)) * NS * BLK

  src = edge_index[0]
  dst = edge_index[1]
  if e_pad != e:
    src = jnp.concatenate([src, jnp.zeros((e_pad - e,), jnp.int32)])
    dst = jnp.concatenate([dst, jnp.full((e_pad - e,), n, jnp.int32)])
  src_g = src.reshape(NW * nbt, AGG_BG)
  dst_g = dst.reshape(NW * nbt, AGG_BG)
  dst_d = dst.reshape(NW * bptd, BLK)
  sc_agg = _make_sc_agg(n, d, nbt, n_pad, jnp.bfloat16)
  sc_deg = _make_sc_deg(bptd, n_pad, d)
  br = 1000 if n % 1000 == 0 else (500 if n % 500 == 0 else n)
  tc_hidden = _make_tc_layer(n, d, n_pad, True, br)
  tc_final = _make_tc_layer(n, d, n_pad, False, br)

  def to_bf16_padded(x):
    return jnp.pad(x.astype(jnp.bfloat16), ((0, n_pad - n), (0, 0)))

  deg = sc_deg(dst_d)
  agg1 = sc_agg(to_bf16_padded(g_features), src_g, dst_g)
  h1 = tc_hidden(g_features, agg1, agg1, deg, deg, W1_self, W1_neigh,
                 b1.reshape(1, d))
  agg2 = sc_agg(to_bf16_padded(h1), src_g, dst_g)
  h2 = tc_hidden(h1, agg2, agg2, deg, deg, W2_self, W2_neigh, b2.reshape(1, d))
  agg3 = sc_agg(to_bf16_padded(h2), src_g, dst_g)
  h3 = tc_final(h2, agg3, agg3, deg, deg, W3_self, W3_neigh, b3.reshape(1, d))
  return h3


# speedup vs baseline: 1.0205x; 1.0205x over previous
import functools
import math

import jax
import jax.numpy as jnp
from jax import lax
from jax.experimental import pallas as pl
from jax.experimental.pallas import tpu as pltpu
from jax.experimental.pallas import tpu_sc as plsc

NC = 2
NS = 16
NW = NC * NS
BLK = 128


AGG_BG = 64
AGG_NB = 4


@functools.lru_cache(maxsize=None)
def _make_sc_agg(n, d, nbt, n_pad, dtype=jnp.float32):
  bg = AGG_BG
  nb = AGG_NB
  rpt = n_pad // NS
  nzc = rpt // BLK
  hb = nbt // 2
  assert hb % nb == 0
  mesh = plsc.VectorSubcoreMesh(core_axis_name="c", subcore_axis_name="s")

  def body(x_hbm, src_hbm, dst_hbm, agg_out, src_stage, dst_stage,
           *rest):
    rows = rest[:nb]
    agg_sh = rest[nb]
    x_sh = rest[nb + 1]
    sems = rest[nb + 2:]
    c = lax.axis_index("c")
    s = lax.axis_index("s")
    w = c * NS + s

    for k in range(nzc):
      r0 = s * rpt + k * BLK
      pltpu.sync_copy(x_hbm.at[pl.ds(r0, BLK), :], x_sh.at[pl.ds(r0, BLK)])

    lane = 16 if dtype == jnp.float32 else 32
    zv = jnp.zeros((lane,), dtype)
    def zrow(i, carry):
      for q in range(d // lane):
        rows[0][i, pl.ds(q * lane, lane)] = zv
      return carry
    lax.fori_loop(0, bg, zrow, 0)
    for k in range(nzc * (BLK // bg)):
      pltpu.sync_copy(rows[0], agg_sh.at[pl.ds(s * rpt + k * bg, bg)])
    plsc.subcore_barrier()

    for h in range(2):
      base = w * nbt + h * hb
      pltpu.sync_copy(src_hbm.at[pl.ds(base, hb), :], src_stage)
      pltpu.sync_copy(dst_hbm.at[pl.ds(base, hb), :], dst_stage)
      for b in range(nb):
        pltpu.async_copy(x_sh.at[src_stage.at[b]], rows[b], sems[b])

      def group(g, carry):
        j0 = nb * g
        for b in range(nb):
          j = j0 + b
          pltpu.make_async_copy(
              x_sh.at[src_stage.at[j]], rows[b], sems[b]).wait()
          pltpu.sync_copy(rows[b], agg_sh.at[dst_stage.at[j]], add=True)

          @pl.when(j + nb < hb)
          def _():
            pltpu.async_copy(
                x_sh.at[src_stage.at[j + nb]], rows[b], sems[b])
        return carry
      lax.fori_loop(0, hb // nb, group, 0)
    plsc.subcore_barrier()

    for k in range(nzc):
      r0 = s * rpt + k * BLK
      pltpu.sync_copy(agg_sh.at[pl.ds(r0, BLK)],
                      agg_out.at[c, pl.ds(r0, BLK), :])

  return pl.kernel(
      body,
      out_type=jax.ShapeDtypeStruct((NC, n_pad, d), dtype),
      mesh=mesh,
      compiler_params=pltpu.CompilerParams(use_tc_tiling_on_sc=False),
      scratch_types=(
          pltpu.VMEM((hb, bg), jnp.int32),
          pltpu.VMEM((hb, bg), jnp.int32),
          *[pltpu.VMEM((bg, d), dtype) for _ in range(nb)],
          pltpu.VMEM_SHARED((n_pad, d), dtype),
          pltpu.VMEM_SHARED((n_pad, d), dtype),
          *[pltpu.SemaphoreType.DMA for _ in range(nb)],
      ),
  )


@functools.lru_cache(maxsize=None)
def _make_sc_deg(bpt, n_pad, d):
  rpt = n_pad // NS
  nzc = rpt // BLK
  mesh = plsc.VectorSubcoreMesh(core_axis_name="c", subcore_axis_name="s")

  def body(dst_hbm, deg_out, dst_stage, obuf, deg_sh):
    c = lax.axis_index("c")
    s = lax.axis_index("s")
    w = c * NS + s

    pltpu.sync_copy(dst_hbm.at[pl.ds(w * bpt, bpt), :], dst_stage)
    zv = jnp.zeros((16,), jnp.float32)
    def zfill(i, carry):
      for q in range(d // 16):
        obuf[i, pl.ds(q * 16, 16)] = zv
      return carry
    lax.fori_loop(0, BLK, zfill, 0)
    for k in range(nzc):
      pltpu.sync_copy(obuf, deg_sh.at[pl.ds(s * rpt + k * BLK, BLK)])
    ov = jnp.ones((16,), jnp.float32)
    def ofill(i, carry):
      for q in range(d // 16):
        obuf[i, pl.ds(q * 16, 16)] = ov
      return carry
    lax.fori_loop(0, BLK, ofill, 0)
    plsc.subcore_barrier()

    def step(j, carry):
      pltpu.sync_copy(obuf, deg_sh.at[dst_stage.at[j]], add=True)
      return carry
    lax.fori_loop(0, bpt, step, 0)
    plsc.subcore_barrier()

    for k in range(nzc):
      r0 = s * rpt + k * BLK
      pltpu.sync_copy(deg_sh.at[pl.ds(r0, BLK)],
                      deg_out.at[c, pl.ds(r0, BLK), :])

  return pl.kernel(
      body,
      out_type=jax.ShapeDtypeStruct((NC, n_pad, d), jnp.float32),
      mesh=mesh,
      compiler_params=pltpu.CompilerParams(use_tc_tiling_on_sc=False),
      scratch_types=(
          pltpu.VMEM((bpt, BLK), jnp.int32),
          pltpu.VMEM((BLK, d), jnp.float32),
          pltpu.VMEM_SHARED((n_pad, d), jnp.float32),
      ),
  )


@functools.lru_cache(maxsize=None)
def _make_tc_layer(n, d, n_pad, relu, br):
  def body(x_ref, a0_ref, a1_ref, d0_ref, d1_ref, ws_ref, wn_ref, b_ref,
           o_ref):
    deg = (d0_ref[0] + d1_ref[0])[:, :1]
    rdeg = 1.0 / jnp.maximum(deg, 1.0)
    hn = (a0_ref[0].astype(jnp.float32) + a1_ref[0].astype(jnp.float32)) * rdeg
    acc = jnp.dot(x_ref[...], ws_ref[...], preferred_element_type=jnp.float32)
    acc = acc + jnp.dot(hn, wn_ref[...], preferred_element_type=jnp.float32)
    acc = acc + b_ref[...]
    o_ref[...] = jnp.maximum(acc, 0.0) if relu else acc

  return pl.pallas_call(
      body,
      grid=(n // br,),
      in_specs=[
          pl.BlockSpec((br, d), lambda i: (i, 0)),
          pl.BlockSpec((1, br, d), lambda i: (0, i, 0)),
          pl.BlockSpec((1, br, d), lambda i: (1, i, 0)),
          pl.BlockSpec((1, br, d), lambda i: (0, i, 0)),
          pl.BlockSpec((1, br, d), lambda i: (1, i, 0)),
          pl.BlockSpec((d, d), lambda i: (0, 0)),
          pl.BlockSpec((d, d), lambda i: (0, 0)),
          pl.BlockSpec((1, d), lambda i: (0, 0)),
      ],
      out_specs=pl.BlockSpec((br, d), lambda i: (i, 0)),
      out_shape=jax.ShapeDtypeStruct((n, d), jnp.float32),
  )


def kernel(g_features, edge_index, W1_self, W1_neigh, b1,
           W2_self, W2_neigh, b2, W3_self, W3_neigh, b3):
  n, d = g_features.shape
  e = edge_index.shape[1]
  quantum = NW * AGG_BG * 2 * AGG_NB
  e_pad = math.ceil(e / quantum) * quantum
  ept = e_pad // NW
  nbt = ept // AGG_BG
  bptd = ept // BLK
  n_pad = math.ceil(n / (NS * BLK)) * NS * BLK

  src = edge_index[0]
  dst = edge_index[1]
  if e_pad != e:
    src = jnp.concatenate([src, jnp.zeros((e_pad - e,), jnp.int32)])
    dst = jnp.concatenate([dst, jnp.full((e_pad - e,), n, jnp.int32)])
  src_g = src.reshape(NW * nbt, AGG_BG)
  dst_g = dst.reshape(NW * nbt, AGG_BG)
  dst_d = dst.reshape(NW * bptd, BLK)
  sc_agg = _make_sc_agg(n, d, nbt, n_pad, jnp.bfloat16)
  sc_deg = _make_sc_deg(bptd, n_pad, d)
  br = 1000 if n % 1000 == 0 else (500 if n % 500 == 0 else n)
  tc_hidden = _make_tc_layer(n, d, n_pad, True, br)
  tc_final = _make_tc_layer(n, d, n_pad, False, br)

  def to_bf16_padded(x):
    return jnp.pad(x.astype(jnp.bfloat16), ((0, n_pad - n), (0, 0)))

  deg = sc_deg(dst_d)
  agg1 = sc_agg(to_bf16_padded(g_features), src_g, dst_g)
  h1 = tc_hidden(g_features, agg1, agg1, deg, deg, W1_self, W1_neigh,
                 b1.reshape(1, d))
  agg2 = sc_agg(to_bf16_padded(h1), src_g, dst_g)
  h2 = tc_hidden(h1, agg2, agg2, deg, deg, W2_self, W2_neigh, b2.reshape(1, d))
  agg3 = sc_agg(to_bf16_padded(h2), src_g, dst_g)
  h3 = tc_final(h2, agg3, agg3, deg, deg, W3_self, W3_neigh, b3.reshape(1, d))
  return h3
